# baseline (speedup 1.0000x reference)
import functools

import jax
import jax.numpy as jnp
from jax import lax
from jax.experimental import pallas as pl
from jax.experimental.pallas import tpu as pltpu
from jax.experimental.pallas import tpu_sc as plsc

N = 10000
NP = 10240
E = 320000
G = 32
HID = 128
NTILE = 16
CHUNK = 64
NCHUNK = 320
IGRP = 32
NGRP = NCHUNK // IGRP
EPT = CHUNK * NCHUNK
EP = EPT * NTILE
RPT = NP // NTILE
LANES = 16


def _proj_body(x_ref, w_ref, b_ref, o_ref, *, act):
    y = lax.dot_general(x_ref[0], w_ref[...], (((1,), (0,)), ((), ())),
                        preferred_element_type=jnp.float32)
    y = y + b_ref[...]
    if act:
        y = jnp.maximum(y, 0.0)
    o_ref[0] = y


def _dense(x, w, b, act, block_rows):
    s, r, k = x.shape
    h = w.shape[1]
    return pl.pallas_call(
        functools.partial(_proj_body, act=act),
        grid=(s, r // block_rows),
        in_specs=[
            pl.BlockSpec((1, block_rows, k), lambda i, j: (i, j, 0)),
            pl.BlockSpec((k, h), lambda i, j: (0, 0)),
            pl.BlockSpec((1, h), lambda i, j: (0, 0)),
        ],
        out_specs=pl.BlockSpec((1, block_rows, h), lambda i, j: (i, j, 0)),
        out_shape=jax.ShapeDtypeStruct((s, r, h), jnp.float32),
    )(x, w, b)


def _mlp_body(h_ref, a_ref, w1_ref, b1_ref, w2_ref, b2_ref, o_ref, *, act):
    z = h_ref[0] + a_ref[0]
    z = lax.dot_general(z, w1_ref[...], (((1,), (0,)), ((), ())),
                        preferred_element_type=jnp.float32) + b1_ref[...]
    z = jnp.maximum(z, 0.0)
    z = lax.dot_general(z, w2_ref[...], (((1,), (0,)), ((), ())),
                        preferred_element_type=jnp.float32) + b2_ref[...]
    if act:
        z = jnp.maximum(z, 0.0)
    o_ref[0] = z


def _mlp(h, agg, w1, b1, w2, b2, act, block_rows=256):
    s, r, hid = h.shape
    return pl.pallas_call(
        functools.partial(_mlp_body, act=act),
        grid=(s, r // block_rows),
        in_specs=[
            pl.BlockSpec((1, block_rows, hid), lambda i, j: (i, j, 0)),
            pl.BlockSpec((1, block_rows, hid), lambda i, j: (i, j, 0)),
            pl.BlockSpec((hid, hid), lambda i, j: (0, 0)),
            pl.BlockSpec((1, hid), lambda i, j: (0, 0)),
            pl.BlockSpec((hid, hid), lambda i, j: (0, 0)),
            pl.BlockSpec((1, hid), lambda i, j: (0, 0)),
        ],
        out_specs=pl.BlockSpec((1, block_rows, hid), lambda i, j: (i, j, 0)),
        out_shape=jax.ShapeDtypeStruct((s, r, hid), jnp.float32),
    )(h, agg, w1, b1, w2, b2)


def _readout_body(batch_ref, h_ref, wp_ref, bp_ref, o_ref, acc_ref):
    s = pl.program_id(0)
    i = pl.program_id(1)

    @pl.when((s == 0) & (i == 0))
    def _():
        acc_ref[...] = jnp.zeros_like(acc_ref)

    b = batch_ref[0, 0, :]
    oh = (b[:, None] == lax.broadcasted_iota(jnp.int32, (b.shape[0], G), 1))
    oh = oh.astype(jnp.float32)
    contrib = lax.dot_general(oh, h_ref[0], (((0,), (0,)), ((), ())),
                              preferred_element_type=jnp.float32)
    sign = jnp.where(s == 0, 1.0, -1.0)
    acc_ref[...] += sign * contrib

    @pl.when((s == pl.num_programs(0) - 1) & (i == pl.num_programs(1) - 1))
    def _():
        o_ref[...] = lax.dot_general(acc_ref[...], wp_ref[...],
                                     (((1,), (0,)), ((), ())),
                                     preferred_element_type=jnp.float32) + bp_ref[...]


def _readout(batch3, h, wp, bp, block_rows=256):
    s, r, hid = h.shape
    nblk = r // block_rows
    return pl.pallas_call(
        _readout_body,
        grid=(s, nblk),
        in_specs=[
            pl.BlockSpec((1, 1, block_rows), lambda i, j: (i * nblk + j, 0, 0)),
            pl.BlockSpec((1, block_rows, hid), lambda i, j: (i, j, 0)),
            pl.BlockSpec((hid, hid), lambda i, j: (0, 0)),
            pl.BlockSpec((1, hid), lambda i, j: (0, 0)),
        ],
        out_specs=pl.BlockSpec((G, hid), lambda i, j: (0, 0)),
        out_shape=jax.ShapeDtypeStruct((G, hid), jnp.float32),
        scratch_shapes=[pltpu.VMEM((G, hid), jnp.float32)],
    )(batch3, h, wp, bp)


def _make_sc_layer():
    mesh = plsc.VectorSubcoreMesh(core_axis_name="c", subcore_axis_name="s")

    @functools.partial(
        pl.kernel,
        mesh=mesh,
        out_type=jax.ShapeDtypeStruct((2 * NP, HID), jnp.float32),
        scratch_types=[
            pltpu.VMEM((2, IGRP, CHUNK), jnp.int32),
            pltpu.VMEM((2, IGRP, CHUNK), jnp.int32),
            pltpu.VMEM((CHUNK, HID), jnp.float32),
            pltpu.VMEM((CHUNK, HID), jnp.float32),
            pltpu.VMEM((CHUNK, HID), jnp.float32),
            pltpu.VMEM((CHUNK, HID), jnp.float32),
            pltpu.VMEM_SHARED((NP, HID), jnp.float32),
            pltpu.SemaphoreType.DMA,
            pltpu.SemaphoreType.DMA,
            pltpu.SemaphoreType.DMA,
            pltpu.SemaphoreType.DMA,
            pltpu.SemaphoreType.DMA,
            pltpu.SemaphoreType.DMA,
        ],
    )
    def sc_layer(h_hbm, e_hbm, src_hbm, dst_hbm, out_hbm,
                 srcg, dstg, hb0, hb1, eb0, eb1, agg,
                 gsem0, gsem1, esem0, esem1, ssem0, ssem1):
        c = lax.axis_index("c")
        t = lax.axis_index("s")
        hbufs = (hb0, hb1)
        ebufs = (eb0, eb1)
        gsems = (gsem0, gsem1)
        esems = (esem0, esem1)
        ssems = (ssem0, ssem1)

        def zrow(r, carry):
            for j in range(HID // LANES):
                z = jnp.zeros((LANES,), jnp.float32)
                hb0[r, pl.ds(j * LANES, LANES)] = z
                eb1[r, pl.ds(j * LANES, LANES)] = z
            return carry
        lax.fori_loop(0, CHUNK, zrow, 0)
        for k in range(RPT // CHUNK):
            pltpu.sync_copy(hb0, agg.at[pl.ds(t * RPT + k * CHUNK, CHUNK)])
        plsc.subcore_barrier()

        def relu_add(hbuf, ebuf):
            def row(rr, carry2):
                for r2 in range(2):
                    r = rr * 2 + r2
                    for j in range(HID // LANES):
                        sl = pl.ds(j * LANES, LANES)
                        ebuf[r, sl] = jnp.maximum(hbuf[r, sl] + ebuf[r, sl], 0.0)
                return carry2
            lax.fori_loop(0, CHUNK // 2, row, 0)

        def issue_loads(jn, hb, eb):
            gn = jn // IGRP
            rn = lax.rem(jn, IGRP)

            @pl.when((rn == 0) & (jn < NCHUNK))
            def _():
                pltpu.sync_copy(src_hbm.at[c, t, gn], srcg.at[lax.rem(gn, 2)])
                pltpu.sync_copy(dst_hbm.at[c, t, gn], dstg.at[lax.rem(gn, 2)])

            jnc = jnp.minimum(jn, NCHUNK - 1)
            gnc = jnc // IGRP
            gh = pltpu.async_copy(h_hbm.at[srcg.at[lax.rem(gnc, 2), lax.rem(jnc, IGRP)]],
                                  hbufs[hb], gsems[hb])
            ge = pltpu.async_copy(e_hbm.at[c, t, jnc], ebufs[eb], esems[eb])
            return gh, ge

        pltpu.sync_copy(src_hbm.at[c, t, 0], srcg.at[0])
        pltpu.sync_copy(dst_hbm.at[c, t, 0], dstg.at[0])
        g0 = pltpu.async_copy(h_hbm.at[srcg.at[0, 0]], hb0, gsem0)
        e0 = pltpu.async_copy(e_hbm.at[c, t, 0], eb0, esem0)
        pltpu.async_copy(eb1, agg.at[dstg.at[0, 0]], ssem1, add=True)
        g0.wait()
        e0.wait()

        def pair(kk, carry):
            for b in range(2):
                j = 2 * kk + b
                pltpu.make_async_copy(ebufs[b ^ 1], agg.at[dstg.at[0, 0]],
                                      ssems[b ^ 1]).wait()
                gh, ge = issue_loads(j + 1, b ^ 1, b ^ 1)
                relu_add(hbufs[b], ebufs[b])
                gj = j // IGRP
                pltpu.async_copy(ebufs[b],
                                 agg.at[dstg.at[lax.rem(gj, 2), lax.rem(j, IGRP)]],
                                 ssems[b], add=True)
                gh.wait()
                ge.wait()
            return carry
        lax.fori_loop(0, NCHUNK // 2, pair, 0)
        pltpu.make_async_copy(eb1, agg.at[dstg.at[0, 0]], ssem1).wait()
        plsc.subcore_barrier()
        pltpu.sync_copy(agg.at[pl.ds(t * RPT, RPT)],
                        out_hbm.at[pl.ds(c * NP + t * RPT, RPT)])

    return sc_layer


_sc_layer = _make_sc_layer()


def kernel(r_x, r_edge_index, r_edge_attr, r_batch,
           p_x, p_edge_index, p_edge_attr, p_batch,
           Wn, bn, We, be, W1, b1, W2, b2, Wp, bp):
    node_in = Wn.shape[0]
    edge_in = We.shape[0]
    ni_pad = 160
    ei_pad = 16

    x = jnp.stack([r_x, p_x])
    x = jnp.pad(x, ((0, 0), (0, NP - N), (0, ni_pad - node_in)))
    wn = jnp.pad(Wn, ((0, ni_pad - node_in), (0, 0)))
    ea = jnp.stack([r_edge_attr, p_edge_attr])
    ea = jnp.pad(ea, ((0, 0), (0, EP - E), (0, ei_pad - edge_in)))
    we = jnp.pad(We, ((0, ei_pad - edge_in), (0, 0)))

    src = jnp.stack([r_edge_index[0], p_edge_index[0]])
    src = jnp.pad(src, ((0, 0), (0, EP - E)))
    src = src + jnp.array([[0], [NP]], jnp.int32)
    dst = jnp.stack([r_edge_index[1], p_edge_index[1]])
    dst = jnp.pad(dst, ((0, 0), (0, EP - E)), constant_values=N)
    src4 = src.reshape(2, NTILE, NGRP, IGRP, CHUNK)
    dst4 = dst.reshape(2, NTILE, NGRP, IGRP, CHUNK)

    batch = jnp.stack([r_batch, p_batch])
    batch = jnp.pad(batch, ((0, 0), (0, NP - N)), constant_values=G)
    batch3 = batch.reshape(2 * (NP // 256), 1, 256)

    bn2 = bn.reshape(1, HID)
    be2 = be.reshape(1, HID)
    wp = jnp.pad(Wp, ((0, 0), (0, HID - Wp.shape[1])))
    bp2 = jnp.pad(bp, ((0, HID - bp.shape[0]),)).reshape(1, HID)

    h = _dense(x, wn, bn2, act=True, block_rows=256)
    e = _dense(ea, we, be2, act=False, block_rows=512)
    e5 = e.reshape(2, NTILE, NCHUNK, CHUNK, HID)

    for i in range(W1.shape[0]):
        agg = _sc_layer(h.reshape(2 * NP, HID), e5, src4, dst4)
        agg = agg.reshape(2, NP, HID)
        h = _mlp(h, agg, W1[i], b1[i].reshape(1, HID),
                 W2[i], b2[i].reshape(1, HID), act=(i < W1.shape[0] - 1))

    out = _readout(batch3, h, wp, bp2)
    return out[:, :Wp.shape[1]]


# speedup vs baseline: 1.0004x; 1.0004x over previous
import functools

import jax
import jax.numpy as jnp
from jax import lax
from jax.experimental import pallas as pl
from jax.experimental.pallas import tpu as pltpu
from jax.experimental.pallas import tpu_sc as plsc

N = 10000
NP = 10240
E = 320000
G = 32
HID = 128
NTILE = 16
CHUNK = 64
NCHUNK = 320
IGRP = 32
NGRP = NCHUNK // IGRP
EPT = CHUNK * NCHUNK
EP = EPT * NTILE
RPT = NP // NTILE
LANES = 16


def _proj_body(x_ref, w_ref, b_ref, o_ref, *, act):
    y = lax.dot_general(x_ref[0], w_ref[...], (((1,), (0,)), ((), ())),
                        preferred_element_type=jnp.float32)
    y = y + b_ref[...]
    if act:
        y = jnp.maximum(y, 0.0)
    o_ref[...] = y


def _dense(x, w, b, act, block_rows):
    s, r, k = x.shape
    h = w.shape[1]
    nblk = r // block_rows
    return pl.pallas_call(
        functools.partial(_proj_body, act=act),
        grid=(s, nblk),
        in_specs=[
            pl.BlockSpec((1, block_rows, k), lambda i, j: (i, j, 0)),
            pl.BlockSpec((k, h), lambda i, j: (0, 0)),
            pl.BlockSpec((1, h), lambda i, j: (0, 0)),
        ],
        out_specs=pl.BlockSpec((block_rows, h),
                               lambda i, j, nblk=nblk: (i * nblk + j, 0)),
        out_shape=jax.ShapeDtypeStruct((s * r, h), jnp.float32),
    )(x, w, b)


def _mlp_body(h_ref, a_ref, w1_ref, b1_ref, w2_ref, b2_ref, o_ref, *, act):
    z = h_ref[...] + a_ref[...]
    z = lax.dot_general(z, w1_ref[...], (((1,), (0,)), ((), ())),
                        preferred_element_type=jnp.float32) + b1_ref[...]
    z = jnp.maximum(z, 0.0)
    z = lax.dot_general(z, w2_ref[...], (((1,), (0,)), ((), ())),
                        preferred_element_type=jnp.float32) + b2_ref[...]
    if act:
        z = jnp.maximum(z, 0.0)
    o_ref[...] = z


def _mlp(h, agg, w1, b1, w2, b2, act, block_rows=256):
    r, hid = h.shape
    return pl.pallas_call(
        functools.partial(_mlp_body, act=act),
        grid=(r // block_rows,),
        in_specs=[
            pl.BlockSpec((block_rows, hid), lambda j: (j, 0)),
            pl.BlockSpec((block_rows, hid), lambda j: (j, 0)),
            pl.BlockSpec((hid, hid), lambda j: (0, 0)),
            pl.BlockSpec((1, hid), lambda j: (0, 0)),
            pl.BlockSpec((hid, hid), lambda j: (0, 0)),
            pl.BlockSpec((1, hid), lambda j: (0, 0)),
        ],
        out_specs=pl.BlockSpec((block_rows, hid), lambda j: (j, 0)),
        out_shape=jax.ShapeDtypeStruct((r, hid), jnp.float32),
    )(h, agg, w1, b1, w2, b2)


def _readout_body(batch_ref, h_ref, wp_ref, bp_ref, o_ref, acc_ref):
    s = pl.program_id(0)
    i = pl.program_id(1)

    @pl.when((s == 0) & (i == 0))
    def _():
        acc_ref[...] = jnp.zeros_like(acc_ref)

    b = batch_ref[0, 0, :]
    oh = (b[:, None] == lax.broadcasted_iota(jnp.int32, (b.shape[0], G), 1))
    oh = oh.astype(jnp.float32)
    contrib = lax.dot_general(oh, h_ref[...], (((0,), (0,)), ((), ())),
                              preferred_element_type=jnp.float32)
    sign = jnp.where(s == 0, 1.0, -1.0)
    acc_ref[...] += sign * contrib

    @pl.when((s == pl.num_programs(0) - 1) & (i == pl.num_programs(1) - 1))
    def _():
        o_ref[...] = lax.dot_general(acc_ref[...], wp_ref[...],
                                     (((1,), (0,)), ((), ())),
                                     preferred_element_type=jnp.float32) + bp_ref[...]


def _readout(batch3, h, wp, bp, block_rows=256):
    r, hid = h.shape
    nblk = r // (2 * block_rows)
    return pl.pallas_call(
        _readout_body,
        grid=(2, nblk),
        in_specs=[
            pl.BlockSpec((1, 1, block_rows),
                         lambda i, j, nblk=nblk: (i * nblk + j, 0, 0)),
            pl.BlockSpec((block_rows, hid),
                         lambda i, j, nblk=nblk: (i * nblk + j, 0)),
            pl.BlockSpec((hid, hid), lambda i, j: (0, 0)),
            pl.BlockSpec((1, hid), lambda i, j: (0, 0)),
        ],
        out_specs=pl.BlockSpec((G, hid), lambda i, j: (0, 0)),
        out_shape=jax.ShapeDtypeStruct((G, hid), jnp.float32),
        scratch_shapes=[pltpu.VMEM((G, hid), jnp.float32)],
    )(batch3, h, wp, bp)


def _make_sc_layer():
    mesh = plsc.VectorSubcoreMesh(core_axis_name="c", subcore_axis_name="s")

    @functools.partial(
        pl.kernel,
        mesh=mesh,
        out_type=jax.ShapeDtypeStruct((2 * NP, HID), jnp.float32),
        scratch_types=[
            pltpu.VMEM((2, IGRP, CHUNK), jnp.int32),
            pltpu.VMEM((2, IGRP, CHUNK), jnp.int32),
            pltpu.VMEM((CHUNK, HID), jnp.float32),
            pltpu.VMEM((CHUNK, HID), jnp.float32),
            pltpu.VMEM((CHUNK, HID), jnp.float32),
            pltpu.VMEM((CHUNK, HID), jnp.float32),
            pltpu.VMEM_SHARED((NP, HID), jnp.float32),
            pltpu.SemaphoreType.DMA,
            pltpu.SemaphoreType.DMA,
            pltpu.SemaphoreType.DMA,
            pltpu.SemaphoreType.DMA,
            pltpu.SemaphoreType.DMA,
            pltpu.SemaphoreType.DMA,
        ],
    )
    def sc_layer(h_hbm, e_hbm, src_hbm, dst_hbm, out_hbm,
                 srcg, dstg, hb0, hb1, eb0, eb1, agg,
                 gsem0, gsem1, esem0, esem1, ssem0, ssem1):
        c = lax.axis_index("c")
        t = lax.axis_index("s")
        hbufs = (hb0, hb1)
        ebufs = (eb0, eb1)
        gsems = (gsem0, gsem1)
        esems = (esem0, esem1)
        ssems = (ssem0, ssem1)

        def zrow(r, carry):
            for j in range(HID // LANES):
                z = jnp.zeros((LANES,), jnp.float32)
                hb0[r, pl.ds(j * LANES, LANES)] = z
                eb1[r, pl.ds(j * LANES, LANES)] = z
            return carry
        lax.fori_loop(0, CHUNK, zrow, 0)
        for k in range(RPT // CHUNK):
            pltpu.sync_copy(hb0, agg.at[pl.ds(t * RPT + k * CHUNK, CHUNK)])
        plsc.subcore_barrier()

        def relu_add(hbuf, ebuf):
            def row(rr, carry2):
                for r2 in range(2):
                    r = rr * 2 + r2
                    for j in range(HID // LANES):
                        sl = pl.ds(j * LANES, LANES)
                        ebuf[r, sl] = jnp.maximum(hbuf[r, sl] + ebuf[r, sl], 0.0)
                return carry2
            lax.fori_loop(0, CHUNK // 2, row, 0)

        def issue_loads(jn, hb, eb):
            gn = jn // IGRP
            rn = lax.rem(jn, IGRP)

            @pl.when((rn == 0) & (jn < NCHUNK))
            def _():
                pltpu.sync_copy(src_hbm.at[c, t, gn], srcg.at[lax.rem(gn, 2)])
                pltpu.sync_copy(dst_hbm.at[c, t, gn], dstg.at[lax.rem(gn, 2)])

            jnc = jnp.minimum(jn, NCHUNK - 1)
            gnc = jnc // IGRP
            gh = pltpu.async_copy(h_hbm.at[srcg.at[lax.rem(gnc, 2), lax.rem(jnc, IGRP)]],
                                  hbufs[hb], gsems[hb])
            ge = pltpu.async_copy(e_hbm.at[c, t, jnc], ebufs[eb], esems[eb])
            return gh, ge

        pltpu.sync_copy(src_hbm.at[c, t, 0], srcg.at[0])
        pltpu.sync_copy(dst_hbm.at[c, t, 0], dstg.at[0])
        g0 = pltpu.async_copy(h_hbm.at[srcg.at[0, 0]], hb0, gsem0)
        e0 = pltpu.async_copy(e_hbm.at[c, t, 0], eb0, esem0)
        pltpu.async_copy(eb1, agg.at[dstg.at[0, 0]], ssem1, add=True)
        g0.wait()
        e0.wait()

        def pair(kk, carry):
            for b in range(2):
                j = 2 * kk + b
                pltpu.make_async_copy(ebufs[b ^ 1], agg.at[dstg.at[0, 0]],
                                      ssems[b ^ 1]).wait()
                gh, ge = issue_loads(j + 1, b ^ 1, b ^ 1)
                relu_add(hbufs[b], ebufs[b])
                gj = j // IGRP
                pltpu.async_copy(ebufs[b],
                                 agg.at[dstg.at[lax.rem(gj, 2), lax.rem(j, IGRP)]],
                                 ssems[b], add=True)
                gh.wait()
                ge.wait()
            return carry
        lax.fori_loop(0, NCHUNK // 2, pair, 0)
        pltpu.make_async_copy(eb1, agg.at[dstg.at[0, 0]], ssem1).wait()
        plsc.subcore_barrier()
        pltpu.sync_copy(agg.at[pl.ds(t * RPT, RPT)],
                        out_hbm.at[pl.ds(c * NP + t * RPT, RPT)])

    return sc_layer


_sc_layer = _make_sc_layer()


def kernel(r_x, r_edge_index, r_edge_attr, r_batch,
           p_x, p_edge_index, p_edge_attr, p_batch,
           Wn, bn, We, be, W1, b1, W2, b2, Wp, bp):
    node_in = Wn.shape[0]
    edge_in = We.shape[0]
    ni_pad = 160
    ei_pad = 16

    x = jnp.stack([r_x, p_x])
    x = jnp.pad(x, ((0, 0), (0, NP - N), (0, ni_pad - node_in)))
    wn = jnp.pad(Wn, ((0, ni_pad - node_in), (0, 0)))
    ea = jnp.stack([r_edge_attr, p_edge_attr])
    ea = jnp.pad(ea, ((0, 0), (0, EP - E), (0, ei_pad - edge_in)))
    we = jnp.pad(We, ((0, ei_pad - edge_in), (0, 0)))

    src = jnp.stack([r_edge_index[0], p_edge_index[0]])
    src = jnp.pad(src, ((0, 0), (0, EP - E)))
    src = src + jnp.array([[0], [NP]], jnp.int32)
    dst = jnp.stack([r_edge_index[1], p_edge_index[1]])
    dst = jnp.pad(dst, ((0, 0), (0, EP - E)), constant_values=N)
    src4 = src.reshape(2, NTILE, NGRP, IGRP, CHUNK)
    dst4 = dst.reshape(2, NTILE, NGRP, IGRP, CHUNK)

    batch = jnp.stack([r_batch, p_batch])
    batch = jnp.pad(batch, ((0, 0), (0, NP - N)), constant_values=G)
    batch3 = batch.reshape(2 * (NP // 256), 1, 256)

    bn2 = bn.reshape(1, HID)
    be2 = be.reshape(1, HID)
    wp = jnp.pad(Wp, ((0, 0), (0, HID - Wp.shape[1])))
    bp2 = jnp.pad(bp, ((0, HID - bp.shape[0]),)).reshape(1, HID)

    h = _dense(x, wn, bn2, act=True, block_rows=256)
    e = _dense(ea, we, be2, act=False, block_rows=512)
    e5 = e.reshape(2, NTILE, NCHUNK, CHUNK, HID)

    for i in range(W1.shape[0]):
        agg = _sc_layer(h, e5, src4, dst4)
        h = _mlp(h, agg, W1[i], b1[i].reshape(1, HID),
                 W2[i], b2[i].reshape(1, HID), act=(i < W1.shape[0] - 1))

    out = _readout(batch3, h, wp, bp2)
    return out[:, :Wp.shape[1]]


# speedup vs baseline: 1.0568x; 1.0564x over previous
import functools

import jax
import jax.numpy as jnp
from jax import lax
from jax.experimental import pallas as pl
from jax.experimental.pallas import tpu as pltpu
from jax.experimental.pallas import tpu_sc as plsc

N = 10000
NP = 10240
E = 320000
G = 32
HID = 128
NTILE = 16
CHUNK = 64
NCHUNK = 320
IGRP = 32
NGRP = NCHUNK // IGRP
EPT = CHUNK * NCHUNK
EP = EPT * NTILE
RPT = NP // NTILE
LANES = 16


def _proj_body(x_ref, w_ref, b_ref, o_ref, *, act):
    y = lax.dot_general(x_ref[0], w_ref[...], (((1,), (0,)), ((), ())),
                        preferred_element_type=jnp.float32)
    y = y + b_ref[...]
    if act:
        y = jnp.maximum(y, 0.0)
    o_ref[...] = y


def _dense(x, w, b, act, block_rows):
    s, r, k = x.shape
    h = w.shape[1]
    nblk = r // block_rows
    return pl.pallas_call(
        functools.partial(_proj_body, act=act),
        grid=(s, nblk),
        in_specs=[
            pl.BlockSpec((1, block_rows, k), lambda i, j: (i, j, 0)),
            pl.BlockSpec((k, h), lambda i, j: (0, 0)),
            pl.BlockSpec((1, h), lambda i, j: (0, 0)),
        ],
        out_specs=pl.BlockSpec((block_rows, h),
                               lambda i, j, nblk=nblk: (i * nblk + j, 0)),
        out_shape=jax.ShapeDtypeStruct((s * r, h), jnp.float32),
    )(x, w, b)


def _edense(ea8, wbig, bbig, block_rows=320):
    s, rp, k = ea8.shape[0], EP // 8, ea8.shape[2]
    n = wbig.shape[1]
    nblk = rp // block_rows
    last = ea8.shape[1] // block_rows - 1
    return pl.pallas_call(
        _proj_body_flat,
        grid=(s, nblk),
        in_specs=[
            pl.BlockSpec((1, block_rows, k),
                         lambda i, j, last=last: (i, jnp.minimum(j, last), 0)),
            pl.BlockSpec((k, n), lambda i, j: (0, 0)),
            pl.BlockSpec((1, n), lambda i, j: (0, 0)),
        ],
        out_specs=pl.BlockSpec((block_rows, n),
                               lambda i, j, nblk=nblk: (i * nblk + j, 0)),
        out_shape=jax.ShapeDtypeStruct((s * rp, n), jnp.float32),
    )(ea8, wbig, bbig)


def _proj_body_flat(x_ref, w_ref, b_ref, o_ref):
    y = lax.dot_general(x_ref[0], w_ref[...], (((1,), (0,)), ((), ())),
                        preferred_element_type=jnp.float32)
    o_ref[...] = y + b_ref[...]


def _mlp_body(h_ref, a_ref, w1_ref, b1_ref, w2_ref, b2_ref, o_ref, *, act):
    z = h_ref[...] + a_ref[...]
    z = lax.dot_general(z, w1_ref[...], (((1,), (0,)), ((), ())),
                        preferred_element_type=jnp.float32) + b1_ref[...]
    z = jnp.maximum(z, 0.0)
    z = lax.dot_general(z, w2_ref[...], (((1,), (0,)), ((), ())),
                        preferred_element_type=jnp.float32) + b2_ref[...]
    if act:
        z = jnp.maximum(z, 0.0)
    o_ref[...] = z


def _mlp(h, agg, w1, b1, w2, b2, act, block_rows=256):
    r, hid = h.shape
    return pl.pallas_call(
        functools.partial(_mlp_body, act=act),
        grid=(r // block_rows,),
        in_specs=[
            pl.BlockSpec((block_rows, hid), lambda j: (j, 0)),
            pl.BlockSpec((block_rows, hid), lambda j: (j, 0)),
            pl.BlockSpec((hid, hid), lambda j: (0, 0)),
            pl.BlockSpec((1, hid), lambda j: (0, 0)),
            pl.BlockSpec((hid, hid), lambda j: (0, 0)),
            pl.BlockSpec((1, hid), lambda j: (0, 0)),
        ],
        out_specs=pl.BlockSpec((block_rows, hid), lambda j: (j, 0)),
        out_shape=jax.ShapeDtypeStruct((r, hid), jnp.float32),
    )(h, agg, w1, b1, w2, b2)


def _readout_body(batch_ref, h_ref, wp_ref, bp_ref, o_ref, acc_ref):
    s = pl.program_id(0)
    i = pl.program_id(1)

    @pl.when((s == 0) & (i == 0))
    def _():
        acc_ref[...] = jnp.zeros_like(acc_ref)

    b = batch_ref[0, 0, :]
    oh = (b[:, None] == lax.broadcasted_iota(jnp.int32, (b.shape[0], G), 1))
    oh = oh.astype(jnp.float32)
    contrib = lax.dot_general(oh, h_ref[...], (((0,), (0,)), ((), ())),
                              preferred_element_type=jnp.float32)
    sign = jnp.where(s == 0, 1.0, -1.0)
    acc_ref[...] += sign * contrib

    @pl.when((s == pl.num_programs(0) - 1) & (i == pl.num_programs(1) - 1))
    def _():
        o_ref[...] = lax.dot_general(acc_ref[...], wp_ref[...],
                                     (((1,), (0,)), ((), ())),
                                     preferred_element_type=jnp.float32) + bp_ref[...]


def _readout(batch3, h, wp, bp, block_rows=256):
    r, hid = h.shape
    nblk = r // (2 * block_rows)
    return pl.pallas_call(
        _readout_body,
        grid=(2, nblk),
        in_specs=[
            pl.BlockSpec((1, 1, block_rows),
                         lambda i, j, nblk=nblk: (i * nblk + j, 0, 0)),
            pl.BlockSpec((block_rows, hid),
                         lambda i, j, nblk=nblk: (i * nblk + j, 0)),
            pl.BlockSpec((hid, hid), lambda i, j: (0, 0)),
            pl.BlockSpec((1, hid), lambda i, j: (0, 0)),
        ],
        out_specs=pl.BlockSpec((G, hid), lambda i, j: (0, 0)),
        out_shape=jax.ShapeDtypeStruct((G, hid), jnp.float32),
        scratch_shapes=[pltpu.VMEM((G, hid), jnp.float32)],
    )(batch3, h, wp, bp)


def _make_sc_layer():
    mesh = plsc.VectorSubcoreMesh(core_axis_name="c", subcore_axis_name="s")

    @functools.partial(
        pl.kernel,
        mesh=mesh,
        out_type=jax.ShapeDtypeStruct((2 * NP, HID), jnp.float32),
        scratch_types=[
            pltpu.VMEM((2, IGRP, CHUNK), jnp.int32),
            pltpu.VMEM((2, IGRP, CHUNK), jnp.int32),
            pltpu.VMEM((CHUNK, HID), jnp.float32),
            pltpu.VMEM((CHUNK, HID), jnp.float32),
            pltpu.VMEM((CHUNK, HID), jnp.float32),
            pltpu.VMEM((CHUNK, HID), jnp.float32),
            pltpu.VMEM_SHARED((NP, HID), jnp.float32),
            pltpu.SemaphoreType.DMA,
            pltpu.SemaphoreType.DMA,
            pltpu.SemaphoreType.DMA,
            pltpu.SemaphoreType.DMA,
            pltpu.SemaphoreType.DMA,
            pltpu.SemaphoreType.DMA,
        ],
    )
    def sc_layer(h_hbm, e_hbm, src_hbm, dst_hbm, out_hbm,
                 srcg, dstg, hb0, hb1, eb0, eb1, agg,
                 gsem0, gsem1, esem0, esem1, ssem0, ssem1):
        c = lax.axis_index("c")
        t = lax.axis_index("s")
        hbufs = (hb0, hb1)
        ebufs = (eb0, eb1)
        gsems = (gsem0, gsem1)
        esems = (esem0, esem1)
        ssems = (ssem0, ssem1)

        def zrow(r, carry):
            for j in range(HID // LANES):
                z = jnp.zeros((LANES,), jnp.float32)
                hb0[r, pl.ds(j * LANES, LANES)] = z
                eb1[r, pl.ds(j * LANES, LANES)] = z
            return carry
        lax.fori_loop(0, CHUNK, zrow, 0)
        for k in range(RPT // CHUNK):
            pltpu.sync_copy(hb0, agg.at[pl.ds(t * RPT + k * CHUNK, CHUNK)])
        plsc.subcore_barrier()

        def relu_add(hbuf, ebuf):
            def row(rr, carry2):
                for r2 in range(2):
                    r = rr * 2 + r2
                    for j in range(HID // LANES):
                        sl = pl.ds(j * LANES, LANES)
                        ebuf[r, sl] = jnp.maximum(hbuf[r, sl] + ebuf[r, sl], 0.0)
                return carry2
            lax.fori_loop(0, CHUNK // 2, row, 0)

        def issue_loads(jn, hb, eb):
            gn = jn // IGRP
            rn = lax.rem(jn, IGRP)

            @pl.when((rn == 0) & (jn < NCHUNK))
            def _():
                pltpu.sync_copy(src_hbm.at[c, t, gn], srcg.at[lax.rem(gn, 2)])
                pltpu.sync_copy(dst_hbm.at[c, t, gn], dstg.at[lax.rem(gn, 2)])

            jnc = jnp.minimum(jn, NCHUNK - 1)
            gnc = jnc // IGRP
            gh = pltpu.async_copy(h_hbm.at[srcg.at[lax.rem(gnc, 2), lax.rem(jnc, IGRP)]],
                                  hbufs[hb], gsems[hb])
            ge = pltpu.async_copy(e_hbm.at[c, t, jnc], ebufs[eb], esems[eb])
            return gh, ge

        pltpu.sync_copy(src_hbm.at[c, t, 0], srcg.at[0])
        pltpu.sync_copy(dst_hbm.at[c, t, 0], dstg.at[0])
        g0 = pltpu.async_copy(h_hbm.at[srcg.at[0, 0]], hb0, gsem0)
        e0 = pltpu.async_copy(e_hbm.at[c, t, 0], eb0, esem0)
        pltpu.async_copy(eb1, agg.at[dstg.at[0, 0]], ssem1, add=True)
        g0.wait()
        e0.wait()

        def pair(kk, carry):
            for b in range(2):
                j = 2 * kk + b
                pltpu.make_async_copy(ebufs[b ^ 1], agg.at[dstg.at[0, 0]],
                                      ssems[b ^ 1]).wait()
                gh, ge = issue_loads(j + 1, b ^ 1, b ^ 1)
                relu_add(hbufs[b], ebufs[b])
                gj = j // IGRP
                pltpu.async_copy(ebufs[b],
                                 agg.at[dstg.at[lax.rem(gj, 2), lax.rem(j, IGRP)]],
                                 ssems[b], add=True)
                gh.wait()
                ge.wait()
            return carry
        lax.fori_loop(0, NCHUNK // 2, pair, 0)
        pltpu.make_async_copy(eb1, agg.at[dstg.at[0, 0]], ssem1).wait()
        plsc.subcore_barrier()
        pltpu.sync_copy(agg.at[pl.ds(t * RPT, RPT)],
                        out_hbm.at[pl.ds(c * NP + t * RPT, RPT)])

    return sc_layer


_sc_layer = _make_sc_layer()


def kernel(r_x, r_edge_index, r_edge_attr, r_batch,
           p_x, p_edge_index, p_edge_attr, p_batch,
           Wn, bn, We, be, W1, b1, W2, b2, Wp, bp):
    node_in = Wn.shape[0]
    edge_in = We.shape[0]
    ni_pad = 160
    ei_pad = 16

    x = jnp.stack([r_x, p_x])
    x = jnp.pad(x, ((0, 0), (0, NP - N), (0, ni_pad - node_in)))
    wn = jnp.pad(Wn, ((0, ni_pad - node_in), (0, 0)))
    ea8 = jnp.stack([r_edge_attr, p_edge_attr]).reshape(2, E // 8, 8 * edge_in)
    wbig = jnp.zeros((8 * edge_in, 8 * HID), jnp.float32)
    for k in range(8):
        wbig = wbig.at[k * edge_in:(k + 1) * edge_in,
                       k * HID:(k + 1) * HID].set(We)
    bbig = jnp.tile(be, 8).reshape(1, 8 * HID)

    src = jnp.stack([r_edge_index[0], p_edge_index[0]])
    src = jnp.pad(src, ((0, 0), (0, EP - E)))
    src = src + jnp.array([[0], [NP]], jnp.int32)
    dst = jnp.stack([r_edge_index[1], p_edge_index[1]])
    dst = jnp.pad(dst, ((0, 0), (0, EP - E)), constant_values=N)
    src4 = src.reshape(2, NTILE, NGRP, IGRP, CHUNK)
    dst4 = dst.reshape(2, NTILE, NGRP, IGRP, CHUNK)

    batch = jnp.stack([r_batch, p_batch])
    batch = jnp.pad(batch, ((0, 0), (0, NP - N)), constant_values=G)
    batch3 = batch.reshape(2 * (NP // 256), 1, 256)

    bn2 = bn.reshape(1, HID)
    wp = jnp.pad(Wp, ((0, 0), (0, HID - Wp.shape[1])))
    bp2 = jnp.pad(bp, ((0, HID - bp.shape[0]),)).reshape(1, HID)

    h = _dense(x, wn, bn2, act=True, block_rows=256)
    e = _edense(ea8, wbig, bbig)
    e5 = e.reshape(2, NTILE, NCHUNK, CHUNK, HID)

    for i in range(W1.shape[0]):
        agg = _sc_layer(h, e5, src4, dst4)
        h = _mlp(h, agg, W1[i], b1[i].reshape(1, HID),
                 W2[i], b2[i].reshape(1, HID), act=(i < W1.shape[0] - 1))

    out = _readout(batch3, h, wp, bp2)
    return out[:, :Wp.shape[1]]


# speedup vs baseline: 1.2523x; 1.1851x over previous
import functools

import jax
import jax.numpy as jnp
from jax import lax
from jax.experimental import pallas as pl
from jax.experimental.pallas import tpu as pltpu
from jax.experimental.pallas import tpu_sc as plsc

N = 10000
NP = 10240
E = 320000
G = 32
HID = 128
NTILE = 16
CHUNK = 64
NCHUNK = 320
IGRP = 32
NGRP = NCHUNK // IGRP
EPT = CHUNK * NCHUNK
EP = EPT * NTILE
RPT = NP // NTILE
LANES = 16


def _proj_body(x_ref, w_ref, b_ref, o_ref, *, act):
    y = lax.dot_general(x_ref[0], w_ref[...], (((1,), (0,)), ((), ())),
                        preferred_element_type=jnp.float32)
    y = y + b_ref[...]
    if act:
        y = jnp.maximum(y, 0.0)
    o_ref[...] = y


def _dense(x, w, b, act, block_rows):
    s, r, k = x.shape
    h = w.shape[1]
    nblk = r // block_rows
    return pl.pallas_call(
        functools.partial(_proj_body, act=act),
        grid=(s, nblk),
        in_specs=[
            pl.BlockSpec((1, block_rows, k), lambda i, j: (i, j, 0)),
            pl.BlockSpec((k, h), lambda i, j: (0, 0)),
            pl.BlockSpec((1, h), lambda i, j: (0, 0)),
        ],
        out_specs=pl.BlockSpec((block_rows, h),
                               lambda i, j, nblk=nblk: (i * nblk + j, 0)),
        out_shape=jax.ShapeDtypeStruct((s * r, h), jnp.float32),
    )(x, w, b)


def _edense(ea8, wbig, bbig, block_rows=320):
    s, rp, k = ea8.shape[0], EP // 8, ea8.shape[2]
    n = wbig.shape[1]
    nblk = rp // block_rows
    nchk = block_rows * 8 // CHUNK
    bpt = NCHUNK // nchk
    last = ea8.shape[1] // block_rows - 1
    return pl.pallas_call(
        _proj_body_5d,
        grid=(s, nblk),
        in_specs=[
            pl.BlockSpec((1, block_rows, k),
                         lambda i, j, last=last: (i, jnp.minimum(j, last), 0)),
            pl.BlockSpec((k, n), lambda i, j: (0, 0)),
            pl.BlockSpec((1, n), lambda i, j: (0, 0)),
        ],
        out_specs=pl.BlockSpec(
            (1, 1, nchk, CHUNK, HID),
            lambda i, j, nchk=nchk, bpt=bpt: (i, j // bpt, j % bpt, 0, 0)),
        out_shape=jax.ShapeDtypeStruct((s, NTILE, NCHUNK, CHUNK, HID),
                                       jnp.float32),
    )(ea8, wbig, bbig)


def _proj_body_5d(x_ref, w_ref, b_ref, o_ref):
    y = lax.dot_general(x_ref[0], w_ref[...], (((1,), (0,)), ((), ())),
                        preferred_element_type=jnp.float32)
    y = y + b_ref[...]
    o_ref[...] = y.reshape(o_ref.shape)


def _mlp_body(h_ref, a_ref, w1_ref, b1_ref, w2_ref, b2_ref, o_ref, *, act):
    z = h_ref[...] + a_ref[...]
    z = lax.dot_general(z, w1_ref[...], (((1,), (0,)), ((), ())),
                        preferred_element_type=jnp.float32) + b1_ref[...]
    z = jnp.maximum(z, 0.0)
    z = lax.dot_general(z, w2_ref[...], (((1,), (0,)), ((), ())),
                        preferred_element_type=jnp.float32) + b2_ref[...]
    if act:
        z = jnp.maximum(z, 0.0)
    o_ref[...] = z


def _mlp(h, agg, w1, b1, w2, b2, act, block_rows=256):
    r, hid = h.shape
    return pl.pallas_call(
        functools.partial(_mlp_body, act=act),
        grid=(r // block_rows,),
        in_specs=[
            pl.BlockSpec((block_rows, hid), lambda j: (j, 0)),
            pl.BlockSpec((block_rows, hid), lambda j: (j, 0)),
            pl.BlockSpec((hid, hid), lambda j: (0, 0)),
            pl.BlockSpec((1, hid), lambda j: (0, 0)),
            pl.BlockSpec((hid, hid), lambda j: (0, 0)),
            pl.BlockSpec((1, hid), lambda j: (0, 0)),
        ],
        out_specs=pl.BlockSpec((block_rows, hid), lambda j: (j, 0)),
        out_shape=jax.ShapeDtypeStruct((r, hid), jnp.float32),
    )(h, agg, w1, b1, w2, b2)


def _readout_body(batch_ref, h_ref, wp_ref, bp_ref, o_ref, acc_ref):
    s = pl.program_id(0)
    i = pl.program_id(1)

    @pl.when((s == 0) & (i == 0))
    def _():
        acc_ref[...] = jnp.zeros_like(acc_ref)

    b = batch_ref[0, 0, :]
    oh = (b[:, None] == lax.broadcasted_iota(jnp.int32, (b.shape[0], G), 1))
    oh = oh.astype(jnp.float32)
    contrib = lax.dot_general(oh, h_ref[...], (((0,), (0,)), ((), ())),
                              preferred_element_type=jnp.float32)
    sign = jnp.where(s == 0, 1.0, -1.0)
    acc_ref[...] += sign * contrib

    @pl.when((s == pl.num_programs(0) - 1) & (i == pl.num_programs(1) - 1))
    def _():
        o_ref[...] = lax.dot_general(acc_ref[...], wp_ref[...],
                                     (((1,), (0,)), ((), ())),
                                     preferred_element_type=jnp.float32) + bp_ref[...]


def _readout(batch3, h, wp, bp, block_rows=256):
    r, hid = h.shape
    nblk = r // (2 * block_rows)
    return pl.pallas_call(
        _readout_body,
        grid=(2, nblk),
        in_specs=[
            pl.BlockSpec((1, 1, block_rows),
                         lambda i, j, nblk=nblk: (i * nblk + j, 0, 0)),
            pl.BlockSpec((block_rows, hid),
                         lambda i, j, nblk=nblk: (i * nblk + j, 0)),
            pl.BlockSpec((hid, hid), lambda i, j: (0, 0)),
            pl.BlockSpec((1, hid), lambda i, j: (0, 0)),
        ],
        out_specs=pl.BlockSpec((G, hid), lambda i, j: (0, 0)),
        out_shape=jax.ShapeDtypeStruct((G, hid), jnp.float32),
        scratch_shapes=[pltpu.VMEM((G, hid), jnp.float32)],
    )(batch3, h, wp, bp)


def _make_sc_layer():
    mesh = plsc.VectorSubcoreMesh(core_axis_name="c", subcore_axis_name="s")

    @functools.partial(
        pl.kernel,
        mesh=mesh,
        out_type=jax.ShapeDtypeStruct((2 * NP, HID), jnp.float32),
        scratch_types=[
            pltpu.VMEM((2, IGRP, CHUNK), jnp.int32),
            pltpu.VMEM((2, IGRP, CHUNK), jnp.int32),
            pltpu.VMEM((CHUNK, HID), jnp.float32),
            pltpu.VMEM((CHUNK, HID), jnp.float32),
            pltpu.VMEM((CHUNK, HID), jnp.float32),
            pltpu.VMEM((CHUNK, HID), jnp.float32),
            pltpu.VMEM_SHARED((NP, HID), jnp.float32),
            pltpu.SemaphoreType.DMA,
            pltpu.SemaphoreType.DMA,
            pltpu.SemaphoreType.DMA,
            pltpu.SemaphoreType.DMA,
            pltpu.SemaphoreType.DMA,
            pltpu.SemaphoreType.DMA,
        ],
    )
    def sc_layer(h_hbm, e_hbm, src_hbm, dst_hbm, out_hbm,
                 srcg, dstg, hb0, hb1, eb0, eb1, agg,
                 gsem0, gsem1, esem0, esem1, ssem0, ssem1):
        c = lax.axis_index("c")
        t = lax.axis_index("s")
        hbufs = (hb0, hb1)
        ebufs = (eb0, eb1)
        gsems = (gsem0, gsem1)
        esems = (esem0, esem1)
        ssems = (ssem0, ssem1)

        def zrow(r, carry):
            for j in range(HID // LANES):
                z = jnp.zeros((LANES,), jnp.float32)
                hb0[r, pl.ds(j * LANES, LANES)] = z
                eb1[r, pl.ds(j * LANES, LANES)] = z
            return carry
        lax.fori_loop(0, CHUNK, zrow, 0)
        for k in range(RPT // CHUNK):
            pltpu.sync_copy(hb0, agg.at[pl.ds(t * RPT + k * CHUNK, CHUNK)])
        plsc.subcore_barrier()

        def relu_add(hbuf, ebuf):
            def row(rr, carry2):
                for r2 in range(2):
                    r = rr * 2 + r2
                    for j in range(HID // LANES):
                        sl = pl.ds(j * LANES, LANES)
                        ebuf[r, sl] = jnp.maximum(hbuf[r, sl] + ebuf[r, sl], 0.0)
                return carry2
            lax.fori_loop(0, CHUNK // 2, row, 0)

        def issue_loads(jn, hb, eb):
            gn = jn // IGRP
            rn = lax.rem(jn, IGRP)

            @pl.when((rn == 0) & (jn < NCHUNK))
            def _():
                pltpu.sync_copy(src_hbm.at[c, t, gn], srcg.at[lax.rem(gn, 2)])
                pltpu.sync_copy(dst_hbm.at[c, t, gn], dstg.at[lax.rem(gn, 2)])

            jnc = jnp.minimum(jn, NCHUNK - 1)
            gnc = jnc // IGRP
            gh = pltpu.async_copy(h_hbm.at[srcg.at[lax.rem(gnc, 2), lax.rem(jnc, IGRP)]],
                                  hbufs[hb], gsems[hb])
            ge = pltpu.async_copy(e_hbm.at[c, t, jnc], ebufs[eb], esems[eb])
            return gh, ge

        pltpu.sync_copy(src_hbm.at[c, t, 0], srcg.at[0])
        pltpu.sync_copy(dst_hbm.at[c, t, 0], dstg.at[0])
        g0 = pltpu.async_copy(h_hbm.at[srcg.at[0, 0]], hb0, gsem0)
        e0 = pltpu.async_copy(e_hbm.at[c, t, 0], eb0, esem0)
        pltpu.async_copy(eb1, agg.at[dstg.at[0, 0]], ssem1, add=True)
        g0.wait()
        e0.wait()

        def pair(kk, carry):
            for b in range(2):
                j = 2 * kk + b
                pltpu.make_async_copy(ebufs[b ^ 1], agg.at[dstg.at[0, 0]],
                                      ssems[b ^ 1]).wait()
                gh, ge = issue_loads(j + 1, b ^ 1, b ^ 1)
                relu_add(hbufs[b], ebufs[b])
                gj = j // IGRP
                pltpu.async_copy(ebufs[b],
                                 agg.at[dstg.at[lax.rem(gj, 2), lax.rem(j, IGRP)]],
                                 ssems[b], add=True)
                gh.wait()
                ge.wait()
            return carry
        lax.fori_loop(0, NCHUNK // 2, pair, 0)
        pltpu.make_async_copy(eb1, agg.at[dstg.at[0, 0]], ssem1).wait()
        plsc.subcore_barrier()
        pltpu.sync_copy(agg.at[pl.ds(t * RPT, RPT)],
                        out_hbm.at[pl.ds(c * NP + t * RPT, RPT)])

    return sc_layer


_sc_layer = _make_sc_layer()


def kernel(r_x, r_edge_index, r_edge_attr, r_batch,
           p_x, p_edge_index, p_edge_attr, p_batch,
           Wn, bn, We, be, W1, b1, W2, b2, Wp, bp):
    node_in = Wn.shape[0]
    edge_in = We.shape[0]
    ni_pad = 160
    ei_pad = 16

    x = jnp.stack([r_x, p_x])
    x = jnp.pad(x, ((0, 0), (0, NP - N), (0, ni_pad - node_in)))
    wn = jnp.pad(Wn, ((0, ni_pad - node_in), (0, 0)))
    ea8 = jnp.stack([r_edge_attr, p_edge_attr]).reshape(2, E // 8, 8 * edge_in)
    wbig = jnp.zeros((8 * edge_in, 8 * HID), jnp.float32)
    for k in range(8):
        wbig = wbig.at[k * edge_in:(k + 1) * edge_in,
                       k * HID:(k + 1) * HID].set(We)
    bbig = jnp.tile(be, 8).reshape(1, 8 * HID)

    src = jnp.stack([r_edge_index[0], p_edge_index[0]])
    src = jnp.pad(src, ((0, 0), (0, EP - E)))
    src = src + jnp.array([[0], [NP]], jnp.int32)
    dst = jnp.stack([r_edge_index[1], p_edge_index[1]])
    dst = jnp.pad(dst, ((0, 0), (0, EP - E)), constant_values=N)
    src4 = src.reshape(2, NTILE, NGRP, IGRP, CHUNK)
    dst4 = dst.reshape(2, NTILE, NGRP, IGRP, CHUNK)

    batch = jnp.stack([r_batch, p_batch])
    batch = jnp.pad(batch, ((0, 0), (0, NP - N)), constant_values=G)
    batch3 = batch.reshape(2 * (NP // 256), 1, 256)

    bn2 = bn.reshape(1, HID)
    wp = jnp.pad(Wp, ((0, 0), (0, HID - Wp.shape[1])))
    bp2 = jnp.pad(bp, ((0, HID - bp.shape[0]),)).reshape(1, HID)

    h = _dense(x, wn, bn2, act=True, block_rows=256)
    e5 = _edense(ea8, wbig, bbig)

    for i in range(W1.shape[0]):
        agg = _sc_layer(h, e5, src4, dst4)
        h = _mlp(h, agg, W1[i], b1[i].reshape(1, HID),
                 W2[i], b2[i].reshape(1, HID), act=(i < W1.shape[0] - 1))

    out = _readout(batch3, h, wp, bp2)
    return out[:, :Wp.shape[1]]
